# baseline (speedup 1.0000x reference)
import functools

import jax
import jax.numpy as jnp
from jax import lax
from jax.experimental import pallas as pl
from jax.experimental.pallas import tpu as pltpu
from jax.experimental.pallas import tpu_sc as plsc

N = 50000
E = 800000
NC = 2
NS = 16
HALF = N // NC
B = 128
ACC_ROWS = 50048
ZROWS = 391
DACC_ROWS = 25008
NBLK_ROW = 1000
NB = N // NBLK_ROW

_SC_PARAMS = pltpu.CompilerParams(use_tc_tiling_on_sc=False,
                                  needs_layout_passes=False)


def _tables_body(x_ref, ws_ref, bias_ref, wd_ref, we_ref, ts_ref, td_ref,
                 te_ref):
    xb = x_ref[...]
    ts_ref[...] = (jnp.dot(xb, ws_ref[0], preferred_element_type=jnp.float32)
                   + bias_ref[0])
    td_ref[...] = jnp.dot(xb, wd_ref[...], preferred_element_type=jnp.float32)
    te_ref[...] = jnp.dot(xb, we_ref[...], preferred_element_type=jnp.float32)


def _tables(xcur, Wstk, bias, Wdst, Wes):
    fin = xcur.shape[1]
    ws = Wstk.shape[2]
    return pl.pallas_call(
        _tables_body,
        grid=(2, NB),
        in_specs=[pl.BlockSpec((NBLK_ROW, fin), lambda j, i: (i, 0)),
                  pl.BlockSpec((1, fin, ws), lambda j, i: (j, 0, 0)),
                  pl.BlockSpec((1, 1, ws), lambda j, i: (j, 0, 0)),
                  pl.BlockSpec((fin, 16), lambda j, i: (0, 0)),
                  pl.BlockSpec((fin, 16), lambda j, i: (0, 0))],
        out_specs=[pl.BlockSpec((NBLK_ROW, ws), lambda j, i: (j * NB + i, 0)),
                   pl.BlockSpec((NBLK_ROW, 16), lambda j, i: (i, 0)),
                   pl.BlockSpec((NBLK_ROW, 16), lambda j, i: (i, 0))],
        out_shape=(jax.ShapeDtypeStruct((2 * N, ws), jnp.float32),
                   jax.ShapeDtypeStruct((N, 16), jnp.float32),
                   jax.ShapeDtypeStruct((N, 16), jnp.float32)),
    )(xcur, Wstk, bias, Wdst, Wes)


def _post_body(two_num, num_ref, num1_ref, den_ref, dm_ref, b_ref, o_ref,
               st_ref, acc):
    i = pl.program_id(0)
    if two_num:
        num = jnp.concatenate([num_ref[...], num1_ref[...]], axis=1)
    else:
        num = num_ref[...]
    dex = jnp.dot(den_ref[...], dm_ref[...], preferred_element_type=jnp.float32)
    o = num / (dex + 1e-16) + b_ref[...]
    o_ref[...] = o

    @pl.when(i == 0)
    def _():
        acc[...] = jnp.zeros_like(acc)

    acc[0:1, :] += jnp.sum(o, axis=0, keepdims=True)
    acc[1:2, :] += jnp.sum(o * o, axis=0, keepdims=True)

    @pl.when(i == NB - 1)
    def _():
        st_ref[...] = acc[0:2, :]


def _post_stats(numarr, den_arr, den_block0, Dm, brow, F, Fc):
    two_num = F == 2 * Fc
    dw = Dm.shape[0]
    return pl.pallas_call(
        functools.partial(_post_body, two_num),
        grid=(NB,),
        in_specs=[pl.BlockSpec((NBLK_ROW, Fc), lambda i: (i, 0)),
                  pl.BlockSpec((NBLK_ROW, Fc), lambda i: (NB + i, 0)),
                  pl.BlockSpec((NBLK_ROW, dw), lambda i: (den_block0 + i, 0)),
                  pl.BlockSpec((dw, F), lambda i: (0, 0)),
                  pl.BlockSpec((1, F), lambda i: (0, 0))],
        out_specs=[pl.BlockSpec((NBLK_ROW, F), lambda i: (i, 0)),
                   pl.BlockSpec((2, F), lambda i: (0, 0))],
        out_shape=(jax.ShapeDtypeStruct((N, F), jnp.float32),
                   jax.ShapeDtypeStruct((2, F), jnp.float32)),
        scratch_shapes=[pltpu.VMEM((8, F), jnp.float32)],
    )(numarr, numarr, den_arr, Dm, brow)


def _norm_tables_body(has_act, o_ref, st_ref, g_ref, bt_ref, m_ref, ws_ref,
                      bias_ref, wd_ref, we_ref, ts_ref, td_ref, te_ref):
    st = st_ref[...]
    mean = st[0:1, :] * (1.0 / N)
    var = st[1:2, :] * (1.0 / N) - mean * mean
    inv = lax.rsqrt(var + 1e-5)
    a = (o_ref[...] - mean) * inv * g_ref[...] + bt_ref[...]
    if has_act:
        a = jnp.where(a > 0, a, jnp.exp(a) - 1.0)
        a = a * jnp.where(m_ref[...] > 0.5, 2.0, 0.0)
    ts_ref[...] = (jnp.dot(a, ws_ref[0], preferred_element_type=jnp.float32)
                   + bias_ref[0])
    td_ref[...] = jnp.dot(a, wd_ref[...], preferred_element_type=jnp.float32)
    te_ref[...] = jnp.dot(a, we_ref[...], preferred_element_type=jnp.float32)


def _norm_tables(o, st, g, bt, mask, Wstk, bias, Wdst, Wes, has_act):
    f = o.shape[1]
    ws = Wstk.shape[2]
    return pl.pallas_call(
        functools.partial(_norm_tables_body, has_act),
        grid=(2, NB),
        in_specs=[pl.BlockSpec((NBLK_ROW, f), lambda j, i: (i, 0)),
                  pl.BlockSpec((2, f), lambda j, i: (0, 0)),
                  pl.BlockSpec((1, f), lambda j, i: (0, 0)),
                  pl.BlockSpec((1, f), lambda j, i: (0, 0)),
                  pl.BlockSpec((NBLK_ROW, f), lambda j, i: (i, 0)),
                  pl.BlockSpec((1, f, ws), lambda j, i: (j, 0, 0)),
                  pl.BlockSpec((1, 1, ws), lambda j, i: (j, 0, 0)),
                  pl.BlockSpec((f, 16), lambda j, i: (0, 0)),
                  pl.BlockSpec((f, 16), lambda j, i: (0, 0))],
        out_specs=[pl.BlockSpec((NBLK_ROW, ws), lambda j, i: (j * NB + i, 0)),
                   pl.BlockSpec((NBLK_ROW, 16), lambda j, i: (i, 0)),
                   pl.BlockSpec((NBLK_ROW, 16), lambda j, i: (i, 0))],
        out_shape=(jax.ShapeDtypeStruct((2 * N, ws), jnp.float32),
                   jax.ShapeDtypeStruct((N, 16), jnp.float32),
                   jax.ShapeDtypeStruct((N, 16), jnp.float32)),
    )(o, st, g, bt, mask, Wstk, bias, Wdst, Wes)


def _final_body(num_ref, den_ref, b_ref, logits_ref, prob_ref):
    actor = num_ref[...] / (den_ref[...] + 1e-16) + b_ref[...]
    t = jnp.tanh(actor)
    logits_ref[...] = t
    m = jnp.max(t, axis=1, keepdims=True)
    p = jnp.exp(t - m)
    prob_ref[...] = p / jnp.sum(p, axis=1, keepdims=True)


def _final(numrow, denrow, brow):
    return pl.pallas_call(
        _final_body,
        out_shape=(jax.ShapeDtypeStruct((1, N), jnp.float32),
                   jax.ShapeDtypeStruct((1, N), jnp.float32)),
    )(numrow, denrow, brow)


def _num_body(Fc, C, nblk, tsrc, tdst, idx3_hbm, zeros_hbm, out,
              idx3_0, idx3_1, dsc0, dsc1, ts_buf0,
              ts_buf1, td_buf0, td_buf1, contrib, ee_rows, sem0, sem1, acc):
    c = lax.axis_index("c")
    s = lax.axis_index("s")
    idx3 = (idx3_0, idx3_1)
    dsc = (dsc0, dsc1)
    ts_buf = (ts_buf0, ts_buf1)
    td_buf = (td_buf0, td_buf1)
    sem = (sem0, sem1)
    heads = [jnp.broadcast_to((c * Fc + j * 16) // C, (16,)).astype(jnp.int32)
             for j in range(Fc // 16)]

    pltpu.sync_copy(zeros_hbm, acc.at[pl.ds(s * 3128, 3128)])
    plsc.subcore_barrier()

    def fetch(blk, p):
        base = (s * nblk + blk) * B
        pltpu.sync_copy(idx3_hbm.at[:, pl.ds(base, B)], idx3[p])
        pltpu.async_copy(tsrc.at[idx3[p].at[c]], ts_buf[p], sem[p])
        pltpu.async_copy(tdst.at[idx3[p].at[2]], td_buf[p], sem[p])

        def di(k2, carry2):
            dsc[p][pl.ds(k2 * 16, 16)] = idx3[p][2, pl.ds(k2 * 16, 16)]
            return carry2

        lax.fori_loop(0, B // 16, di, 0, unroll=True)

    def crunch(p):
        pltpu.make_async_copy(tsrc.at[pl.ds(0, B)], ts_buf[p], sem[p]).wait()
        pltpu.make_async_copy(tdst.at[pl.ds(0, B)], td_buf[p], sem[p]).wait()

        @plsc.parallel_loop(0, B, unroll=8)
        def eb(b2):
            e = ts_buf[p][b2, pl.ds(0, 16)] + td_buf[p][b2, pl.ds(0, 16)]
            e = jnp.maximum(e, 0.2 * e)
            ee_rows[b2, pl.ds(0, 16)] = jnp.exp(e)
            row = jnp.broadcast_to(b2, (16,)).astype(jnp.int32)
            for j in range(Fc // 16):
                sp = plsc.load_gather(ee_rows, [row, heads[j]])
                contrib[b2, pl.ds(j * 16, 16)] = (
                    sp * ts_buf[p][b2, pl.ds(16 + j * 16, 16)])

        pltpu.sync_copy(contrib, acc.at[dsc[p]], add=True)

    fetch(0, 0)

    def blk_body(i, carry):
        for p in range(2):
            blk = 2 * i + p

            @pl.when(blk < nblk - 1)
            def _():
                fetch(blk + 1, 1 - p)

            crunch(p)
        return carry

    lax.fori_loop(0, nblk // 2, blk_body, 0)
    plsc.subcore_barrier()
    pltpu.sync_copy(acc.at[pl.ds(s * 3125, 3125)],
                    out.at[pl.ds(c * N + s * 3125, 3125)])


def _make_num_kernel(E_pad, Fc, C):
    ws = 16 + Fc
    nblk = E_pad // (NS * B)
    mesh = plsc.VectorSubcoreMesh(core_axis_name="c", subcore_axis_name="s",
                                  num_cores=NC, num_subcores=NS)
    return pl.kernel(
        functools.partial(_num_body, Fc, C, nblk),
        out_type=jax.ShapeDtypeStruct((2 * N, Fc), jnp.float32),
        mesh=mesh,
        compiler_params=_SC_PARAMS,
        scratch_types=[
            pltpu.VMEM((3, B), jnp.int32),
            pltpu.VMEM((3, B), jnp.int32),
            pltpu.VMEM((B,), jnp.int32),
            pltpu.VMEM((B,), jnp.int32),
            pltpu.VMEM((B, ws), jnp.float32),
            pltpu.VMEM((B, ws), jnp.float32),
            pltpu.VMEM((B, 16), jnp.float32),
            pltpu.VMEM((B, 16), jnp.float32),
            pltpu.VMEM((B, Fc), jnp.float32),
            pltpu.VMEM((B, 16), jnp.float32),
            pltpu.SemaphoreType.DMA,
            pltpu.SemaphoreType.DMA,
            pltpu.VMEM_SHARED((ACC_ROWS, Fc), jnp.float32),
        ],
    )


def _den_body(H, nblk, tes, tdst, idx3_hbm, zeros_hbm, out, idx3_0, idx3_1,
              loc_idx0, loc_idx1, ts_buf0,
              ts_buf1, td_buf0, td_buf1, denrow, sem0, sem1, acc):
    c = lax.axis_index("c")
    s = lax.axis_index("s")
    iota = lax.iota(jnp.int32, 16)
    idx3 = (idx3_0, idx3_1)
    loc_idx = (loc_idx0, loc_idx1)
    ts_buf = (ts_buf0, ts_buf1)
    td_buf = (td_buf0, td_buf1)
    sem = (sem0, sem1)

    pltpu.sync_copy(zeros_hbm, acc.at[pl.ds(s * 1563, 1563)])
    plsc.subcore_barrier()

    def fetch(blk, p):
        base = (s * nblk + blk) * B
        pltpu.sync_copy(idx3_hbm.at[:, pl.ds(base, B)], idx3[p])
        pltpu.async_copy(tes.at[idx3[p].at[0]], ts_buf[p], sem[p])
        pltpu.async_copy(tdst.at[idx3[p].at[2]], td_buf[p], sem[p])

        def li(k2, carry2):
            d = idx3[p][2, pl.ds(k2 * 16, 16)]
            l = d - c * HALF
            ok = (l >= 0) & (l < HALF)
            loc_idx[p][pl.ds(k2 * 16, 16)] = jnp.where(ok, l, HALF)
            return carry2

        lax.fori_loop(0, B // 16, li, 0, unroll=True)

    def crunch(p):
        pltpu.make_async_copy(tes.at[pl.ds(0, B)], ts_buf[p], sem[p]).wait()
        pltpu.make_async_copy(tdst.at[pl.ds(0, B)], td_buf[p], sem[p]).wait()

        @plsc.parallel_loop(0, B, unroll=8)
        def eb(b2):
            e = ts_buf[p][b2, pl.ds(0, 16)] + td_buf[p][b2, pl.ds(0, 16)]
            e = jnp.maximum(e, 0.2 * e)
            denrow[b2, pl.ds(0, 16)] = jnp.where(iota < H, jnp.exp(e), 0.0)

        pltpu.sync_copy(denrow, acc.at[loc_idx[p]], add=True)

    fetch(0, 0)

    def blk_body(i, carry):
        for p in range(2):
            blk = 2 * i + p

            @pl.when(blk < nblk - 1)
            def _():
                fetch(blk + 1, 1 - p)

            crunch(p)
        return carry

    lax.fori_loop(0, nblk // 2, blk_body, 0)
    plsc.subcore_barrier()
    start = s * 1563 - jnp.maximum(s - 8, 0)

    @pl.when(s < 8)
    def _():
        pltpu.sync_copy(acc.at[pl.ds(start, 1563)],
                        out.at[pl.ds(c * HALF + start, 1563)])

    @pl.when(s >= 8)
    def _():
        pltpu.sync_copy(acc.at[pl.ds(start, 1562)],
                        out.at[pl.ds(c * HALF + start, 1562)])


def _make_den_kernel(E_pad, H):
    nblk = E_pad // (NS * B)
    mesh = plsc.VectorSubcoreMesh(core_axis_name="c", subcore_axis_name="s",
                                  num_cores=NC, num_subcores=NS)
    return pl.kernel(
        functools.partial(_den_body, H, nblk),
        out_type=jax.ShapeDtypeStruct((N, 16), jnp.float32),
        mesh=mesh,
        compiler_params=_SC_PARAMS,
        scratch_types=[
            pltpu.VMEM((3, B), jnp.int32),
            pltpu.VMEM((3, B), jnp.int32),
            pltpu.VMEM((B,), jnp.int32),
            pltpu.VMEM((B,), jnp.int32),
            pltpu.VMEM((B, 16), jnp.float32),
            pltpu.VMEM((B, 16), jnp.float32),
            pltpu.VMEM((B, 16), jnp.float32),
            pltpu.VMEM((B, 16), jnp.float32),
            pltpu.VMEM((B, 16), jnp.float32),
            pltpu.SemaphoreType.DMA,
            pltpu.SemaphoreType.DMA,
            pltpu.VMEM_SHARED((DACC_ROWS, 16), jnp.float32),
        ],
    )


def _fold4(W, a_s, a_d):
    F = W.shape[1]
    eyeH = jnp.eye(4, dtype=jnp.float32)
    Ss = (a_s[:, :, None] * eyeH[:, None, :]).reshape(F, 4)
    Sd = (a_d[:, :, None] * eyeH[:, None, :]).reshape(F, 4)
    Wes = jnp.pad(W @ Ss, ((0, 0), (0, 12)))
    Wed = jnp.pad(W @ Sd, ((0, 0), (0, 12)))
    Wstk = jnp.stack([jnp.concatenate([Wes, W[:, 0:32]], axis=1),
                      jnp.concatenate([Wes, W[:, 32:64]], axis=1)])
    bias = jnp.zeros((2, 1, 48), jnp.float32)
    return Wstk, bias, Wed, Wes


def _fold1(W, a_s, a_d, Fc):
    fin = W.shape[0]
    Wes = (W @ a_s.T) @ jnp.ones((1, 16), jnp.float32)
    Wed = (W @ a_d.T) @ jnp.ones((1, 16), jnp.float32)
    hpad = jnp.pad(W, ((0, 0), (0, Fc - W.shape[1])))
    Wstk = jnp.stack([jnp.concatenate([Wes, hpad], axis=1),
                      jnp.concatenate([Wes, jnp.zeros((fin, Fc), jnp.float32)],
                                      axis=1)])
    maskrow = jnp.concatenate(
        [jnp.zeros((1, 16), jnp.float32),
         (lax.iota(jnp.int32, Fc) < 1)[None].astype(jnp.float32)], axis=1)
    bias = jnp.stack([jnp.zeros((1, 16 + Fc), jnp.float32), maskrow])
    return Wstk, bias, Wed, Wes


def _dm4():
    Dm = jnp.kron(jnp.eye(4, dtype=jnp.float32), jnp.ones((1, 16), jnp.float32))
    return jnp.pad(Dm, ((0, 12), (0, 0)))


def kernel(x, edge_index, W1, as1, ad1, b1, W2, as2, ad2, b2, W3, as3, ad3, b3,
           W4, as4, ad4, b4, g1, bt1, g2, bt2, g3, bt3, mask1, mask2):
    src, dst = edge_index[0], edge_index[1]
    nblk = (E + NS * B - 1) // (NS * B)
    nblk += nblk % 2
    e_pad = nblk * NS * B
    pad = e_pad - E
    src_p = jnp.concatenate([src, jnp.zeros((pad,), jnp.int32)])
    dst_p = jnp.concatenate([dst, jnp.full((pad,), N, jnp.int32)])
    idx3 = jnp.stack([src_p, src_p + N, dst_p])
    z32 = jnp.zeros((3128, 32), jnp.float32)
    z16n = jnp.zeros((3128, 16), jnp.float32)
    z16d = jnp.zeros((1563, 16), jnp.float32)

    num64 = _make_num_kernel(e_pad, 32, 16)
    den4 = _make_den_kernel(e_pad, 4)
    num32 = _make_num_kernel(e_pad, 32, 32)
    num16 = _make_num_kernel(e_pad, 16, 16)

    Wstk, bias, Wed, Wes = _fold4(W1, as1, ad1)
    ts, td, te = _tables(x, Wstk, bias, Wed, Wes)
    numarr = num64(ts, td, idx3, z32)
    den = den4(te, td, idx3, z16d)
    o, st = _post_stats(numarr, den, 0, _dm4(), b1.reshape(1, -1), 64, 32)

    Wstk, bias, Wed, Wes = _fold4(W2, as2, ad2)
    ts, td, te = _norm_tables(o, st, g1.reshape(1, -1), bt1.reshape(1, -1),
                              mask1, Wstk, bias, Wed, Wes, True)
    numarr = num64(ts, td, idx3, z32)
    den = den4(te, td, idx3, z16d)
    o, st = _post_stats(numarr, den, 0, _dm4(), b2.reshape(1, -1), 64, 32)

    Wstk, bias, Wed, Wes = _fold1(W3, as3, ad3, 32)
    ts, td, te = _norm_tables(o, st, g2.reshape(1, -1), bt2.reshape(1, -1),
                              mask2, Wstk, bias, Wed, Wes, True)
    numarr = num32(ts, td, idx3, z32)
    dm3 = jnp.zeros((32, 32), jnp.float32).at[0, :].set(1.0)
    o, st = _post_stats(numarr, numarr, NB, dm3, b3.reshape(1, -1), 32, 32)

    W4p = jnp.pad(W4, ((0, 0), (0, 15)))
    as4p = jnp.pad(as4, ((0, 0), (0, 15)))
    ad4p = jnp.pad(ad4, ((0, 0), (0, 15)))
    Wstk, bias, Wed, Wes = _fold1(W4p, as4p, ad4p, 16)
    ts, td, te = _norm_tables(o, st, g3.reshape(1, -1), bt3.reshape(1, -1), o,
                              Wstk, bias, Wed, Wes, False)
    numarr = num16(ts, td, idx3, z16n)

    numrow = numarr[0:N, 0:1].reshape(1, N)
    denrow = numarr[N:2 * N, 0:1].reshape(1, N)
    logits, prob = _final(numrow, denrow, b4.reshape(1, 1))
    return (logits, prob)


# speedup vs baseline: 1.1184x; 1.1184x over previous
import functools

import jax
import jax.numpy as jnp
from jax import lax
from jax.experimental import pallas as pl
from jax.experimental.pallas import tpu as pltpu
from jax.experimental.pallas import tpu_sc as plsc

N = 50000
E = 800000
NC = 2
NS = 16
HALF = N // NC
B = 128
ACC_ROWS = 50048
ZROWS = 391
DACC_ROWS = 25008
NBLK_ROW = 1000
NB = N // NBLK_ROW

_SC_PARAMS = pltpu.CompilerParams(use_tc_tiling_on_sc=False,
                                  needs_layout_passes=False)


def _tables_body(x_ref, ws_ref, bias_ref, wd_ref, we_ref, ts_ref, td_ref,
                 te_ref):
    xb = x_ref[...]
    ts_ref[...] = (jnp.dot(xb, ws_ref[0], preferred_element_type=jnp.float32)
                   + bias_ref[0])
    td_ref[...] = jnp.dot(xb, wd_ref[...], preferred_element_type=jnp.float32)
    te_ref[...] = jnp.dot(xb, we_ref[...], preferred_element_type=jnp.float32)


def _tables(xcur, Wstk, bias, Wdst, Wes):
    fin = xcur.shape[1]
    ws = Wstk.shape[2]
    return pl.pallas_call(
        _tables_body,
        grid=(2, NB),
        in_specs=[pl.BlockSpec((NBLK_ROW, fin), lambda j, i: (i, 0)),
                  pl.BlockSpec((1, fin, ws), lambda j, i: (j, 0, 0)),
                  pl.BlockSpec((1, 1, ws), lambda j, i: (j, 0, 0)),
                  pl.BlockSpec((fin, 16), lambda j, i: (0, 0)),
                  pl.BlockSpec((fin, 16), lambda j, i: (0, 0))],
        out_specs=[pl.BlockSpec((NBLK_ROW, ws), lambda j, i: (j * NB + i, 0)),
                   pl.BlockSpec((NBLK_ROW, 16), lambda j, i: (i, 0)),
                   pl.BlockSpec((NBLK_ROW, 16), lambda j, i: (i, 0))],
        out_shape=(jax.ShapeDtypeStruct((2 * N, ws), jnp.float32),
                   jax.ShapeDtypeStruct((N, 16), jnp.float32),
                   jax.ShapeDtypeStruct((N, 16), jnp.float32)),
    )(xcur, Wstk, bias, Wdst, Wes)


def _post_body(two_num, num_ref, num1_ref, den_ref, dm_ref, b_ref, o_ref,
               st_ref, acc):
    i = pl.program_id(0)
    if two_num:
        num = jnp.concatenate([num_ref[...], num1_ref[...]], axis=1)
    else:
        num = num_ref[...]
    dex = jnp.dot(den_ref[...], dm_ref[...], preferred_element_type=jnp.float32)
    o = num / (dex + 1e-16) + b_ref[...]
    o_ref[...] = o

    @pl.when(i == 0)
    def _():
        acc[...] = jnp.zeros_like(acc)

    acc[0:1, :] += jnp.sum(o, axis=0, keepdims=True)
    acc[1:2, :] += jnp.sum(o * o, axis=0, keepdims=True)

    @pl.when(i == NB - 1)
    def _():
        st_ref[...] = acc[0:2, :]


def _post_stats(numarr, den_arr, den_block0, Dm, brow, F, Fc):
    two_num = F == 2 * Fc
    dw = Dm.shape[0]
    return pl.pallas_call(
        functools.partial(_post_body, two_num),
        grid=(NB,),
        in_specs=[pl.BlockSpec((NBLK_ROW, Fc), lambda i: (i, 0)),
                  pl.BlockSpec((NBLK_ROW, Fc), lambda i: (NB + i, 0)),
                  pl.BlockSpec((NBLK_ROW, dw), lambda i: (den_block0 + i, 0)),
                  pl.BlockSpec((dw, F), lambda i: (0, 0)),
                  pl.BlockSpec((1, F), lambda i: (0, 0))],
        out_specs=[pl.BlockSpec((NBLK_ROW, F), lambda i: (i, 0)),
                   pl.BlockSpec((2, F), lambda i: (0, 0))],
        out_shape=(jax.ShapeDtypeStruct((N, F), jnp.float32),
                   jax.ShapeDtypeStruct((2, F), jnp.float32)),
        scratch_shapes=[pltpu.VMEM((8, F), jnp.float32)],
    )(numarr, numarr, den_arr, Dm, brow)


def _norm_tables_body(has_act, o_ref, st_ref, g_ref, bt_ref, m_ref, ws_ref,
                      bias_ref, wd_ref, we_ref, ts_ref, td_ref, te_ref):
    st = st_ref[...]
    mean = st[0:1, :] * (1.0 / N)
    var = st[1:2, :] * (1.0 / N) - mean * mean
    inv = lax.rsqrt(var + 1e-5)
    a = (o_ref[...] - mean) * inv * g_ref[...] + bt_ref[...]
    if has_act:
        a = jnp.where(a > 0, a, jnp.exp(a) - 1.0)
        a = a * jnp.where(m_ref[...] > 0.5, 2.0, 0.0)
    ts_ref[...] = (jnp.dot(a, ws_ref[0], preferred_element_type=jnp.float32)
                   + bias_ref[0])
    td_ref[...] = jnp.dot(a, wd_ref[...], preferred_element_type=jnp.float32)
    te_ref[...] = jnp.dot(a, we_ref[...], preferred_element_type=jnp.float32)


def _norm_tables(o, st, g, bt, mask, Wstk, bias, Wdst, Wes, has_act):
    f = o.shape[1]
    ws = Wstk.shape[2]
    return pl.pallas_call(
        functools.partial(_norm_tables_body, has_act),
        grid=(2, NB),
        in_specs=[pl.BlockSpec((NBLK_ROW, f), lambda j, i: (i, 0)),
                  pl.BlockSpec((2, f), lambda j, i: (0, 0)),
                  pl.BlockSpec((1, f), lambda j, i: (0, 0)),
                  pl.BlockSpec((1, f), lambda j, i: (0, 0)),
                  pl.BlockSpec((NBLK_ROW, f), lambda j, i: (i, 0)),
                  pl.BlockSpec((1, f, ws), lambda j, i: (j, 0, 0)),
                  pl.BlockSpec((1, 1, ws), lambda j, i: (j, 0, 0)),
                  pl.BlockSpec((f, 16), lambda j, i: (0, 0)),
                  pl.BlockSpec((f, 16), lambda j, i: (0, 0))],
        out_specs=[pl.BlockSpec((NBLK_ROW, ws), lambda j, i: (j * NB + i, 0)),
                   pl.BlockSpec((NBLK_ROW, 16), lambda j, i: (i, 0)),
                   pl.BlockSpec((NBLK_ROW, 16), lambda j, i: (i, 0))],
        out_shape=(jax.ShapeDtypeStruct((2 * N, ws), jnp.float32),
                   jax.ShapeDtypeStruct((N, 16), jnp.float32),
                   jax.ShapeDtypeStruct((N, 16), jnp.float32)),
    )(o, st, g, bt, mask, Wstk, bias, Wdst, Wes)


def _final_body(num_ref, den_ref, b_ref, logits_ref, prob_ref):
    actor = num_ref[...] / (den_ref[...] + 1e-16) + b_ref[...]
    t = jnp.tanh(actor)
    logits_ref[...] = t
    m = jnp.max(t, axis=1, keepdims=True)
    p = jnp.exp(t - m)
    prob_ref[...] = p / jnp.sum(p, axis=1, keepdims=True)


def _final(numrow, denrow, brow):
    return pl.pallas_call(
        _final_body,
        out_shape=(jax.ShapeDtypeStruct((1, N), jnp.float32),
                   jax.ShapeDtypeStruct((1, N), jnp.float32)),
    )(numrow, denrow, brow)


def _num_body(Fc, C, nblk, tsrc, tdst, idx3_hbm, zeros_hbm, out,
              idx3_0, idx3_1, idx3_2, idx3_3, dsc0, dsc1, ts_buf0,
              ts_buf1, td_buf0, td_buf1, contrib, ee_rows, sem0, sem1,
              semi0, semi1, semi2, semi3, acc):
    c = lax.axis_index("c")
    s = lax.axis_index("s")
    idx3 = (idx3_0, idx3_1, idx3_2, idx3_3)
    dsc = (dsc0, dsc1)
    ts_buf = (ts_buf0, ts_buf1)
    td_buf = (td_buf0, td_buf1)
    sem = (sem0, sem1)
    semi = (semi0, semi1, semi2, semi3)
    heads = [jnp.broadcast_to((c * Fc + j * 16) // C, (16,)).astype(jnp.int32)
             for j in range(Fc // 16)]

    pltpu.sync_copy(zeros_hbm, acc.at[pl.ds(s * 3128, 3128)])
    plsc.subcore_barrier()

    def issue_idx(blk, q):
        base = (s * nblk + blk) * B
        pltpu.async_copy(idx3_hbm.at[:, pl.ds(base, B)], idx3[q], semi[q])

    def wait_idx(q):
        pltpu.make_async_copy(idx3_hbm.at[:, pl.ds(0, B)], idx3[q],
                              semi[q]).wait()

    def issue_gathers(p, q):
        pltpu.async_copy(tsrc.at[idx3[q].at[c]], ts_buf[p], sem[p])
        pltpu.async_copy(tdst.at[idx3[q].at[2]], td_buf[p], sem[p])

        def di(k2, carry2):
            dsc[p][pl.ds(k2 * 16, 16)] = idx3[q][2, pl.ds(k2 * 16, 16)]
            return carry2

        lax.fori_loop(0, B // 16, di, 0, unroll=True)

    def crunch(p):
        pltpu.make_async_copy(tsrc.at[pl.ds(0, B)], ts_buf[p], sem[p]).wait()
        pltpu.make_async_copy(tdst.at[pl.ds(0, B)], td_buf[p], sem[p]).wait()

        @plsc.parallel_loop(0, B, unroll=8)
        def eb(b2):
            e = ts_buf[p][b2, pl.ds(0, 16)] + td_buf[p][b2, pl.ds(0, 16)]
            e = jnp.maximum(e, 0.2 * e)
            ee_rows[b2, pl.ds(0, 16)] = jnp.exp(e)
            row = jnp.broadcast_to(b2, (16,)).astype(jnp.int32)
            for j in range(Fc // 16):
                sp = plsc.load_gather(ee_rows, [row, heads[j]])
                contrib[b2, pl.ds(j * 16, 16)] = (
                    sp * ts_buf[p][b2, pl.ds(16 + j * 16, 16)])

        pltpu.sync_copy(contrib, acc.at[dsc[p]], add=True)

    issue_idx(0, 0)
    issue_idx(1, 1)
    wait_idx(0)
    issue_gathers(0, 0)

    def blk_body(i, carry):
        for p4 in range(4):
            blk = 4 * i + p4

            @pl.when(blk + 1 < nblk)
            def _():
                wait_idx((p4 + 1) % 4)
                issue_gathers((p4 + 1) % 2, (p4 + 1) % 4)

            @pl.when(blk + 2 < nblk)
            def _():
                issue_idx(blk + 2, (p4 + 2) % 4)

            crunch(p4 % 2)
        return carry

    lax.fori_loop(0, nblk // 4, blk_body, 0)
    plsc.subcore_barrier()
    pltpu.sync_copy(acc.at[pl.ds(s * 3125, 3125)],
                    out.at[pl.ds(c * N + s * 3125, 3125)])


def _make_num_kernel(E_pad, Fc, C):
    ws = 16 + Fc
    nblk = E_pad // (NS * B)
    mesh = plsc.VectorSubcoreMesh(core_axis_name="c", subcore_axis_name="s",
                                  num_cores=NC, num_subcores=NS)
    return pl.kernel(
        functools.partial(_num_body, Fc, C, nblk),
        out_type=jax.ShapeDtypeStruct((2 * N, Fc), jnp.float32),
        mesh=mesh,
        compiler_params=_SC_PARAMS,
        scratch_types=[
            pltpu.VMEM((3, B), jnp.int32),
            pltpu.VMEM((3, B), jnp.int32),
            pltpu.VMEM((3, B), jnp.int32),
            pltpu.VMEM((3, B), jnp.int32),
            pltpu.VMEM((B,), jnp.int32),
            pltpu.VMEM((B,), jnp.int32),
            pltpu.VMEM((B, ws), jnp.float32),
            pltpu.VMEM((B, ws), jnp.float32),
            pltpu.VMEM((B, 16), jnp.float32),
            pltpu.VMEM((B, 16), jnp.float32),
            pltpu.VMEM((B, Fc), jnp.float32),
            pltpu.VMEM((B, 16), jnp.float32),
            pltpu.SemaphoreType.DMA,
            pltpu.SemaphoreType.DMA,
            pltpu.SemaphoreType.DMA,
            pltpu.SemaphoreType.DMA,
            pltpu.SemaphoreType.DMA,
            pltpu.SemaphoreType.DMA,
            pltpu.VMEM_SHARED((ACC_ROWS, Fc), jnp.float32),
        ],
    )


def _den_body(H, nblk, tes, tdst, idx3_hbm, zeros_hbm, out, idx3_0, idx3_1,
              loc_idx0, loc_idx1, ts_buf0,
              ts_buf1, td_buf0, td_buf1, denrow, sem0, sem1, acc):
    c = lax.axis_index("c")
    s = lax.axis_index("s")
    iota = lax.iota(jnp.int32, 16)
    idx3 = (idx3_0, idx3_1)
    loc_idx = (loc_idx0, loc_idx1)
    ts_buf = (ts_buf0, ts_buf1)
    td_buf = (td_buf0, td_buf1)
    sem = (sem0, sem1)

    pltpu.sync_copy(zeros_hbm, acc.at[pl.ds(s * 1563, 1563)])
    plsc.subcore_barrier()

    def fetch(blk, p):
        base = (s * nblk + blk) * B
        pltpu.sync_copy(idx3_hbm.at[:, pl.ds(base, B)], idx3[p])
        pltpu.async_copy(tes.at[idx3[p].at[0]], ts_buf[p], sem[p])
        pltpu.async_copy(tdst.at[idx3[p].at[2]], td_buf[p], sem[p])

        def li(k2, carry2):
            d = idx3[p][2, pl.ds(k2 * 16, 16)]
            l = d - c * HALF
            ok = (l >= 0) & (l < HALF)
            loc_idx[p][pl.ds(k2 * 16, 16)] = jnp.where(ok, l, HALF)
            return carry2

        lax.fori_loop(0, B // 16, li, 0, unroll=True)

    def crunch(p):
        pltpu.make_async_copy(tes.at[pl.ds(0, B)], ts_buf[p], sem[p]).wait()
        pltpu.make_async_copy(tdst.at[pl.ds(0, B)], td_buf[p], sem[p]).wait()

        @plsc.parallel_loop(0, B, unroll=8)
        def eb(b2):
            e = ts_buf[p][b2, pl.ds(0, 16)] + td_buf[p][b2, pl.ds(0, 16)]
            e = jnp.maximum(e, 0.2 * e)
            denrow[b2, pl.ds(0, 16)] = jnp.where(iota < H, jnp.exp(e), 0.0)

        pltpu.sync_copy(denrow, acc.at[loc_idx[p]], add=True)

    fetch(0, 0)

    def blk_body(i, carry):
        for p in range(2):
            blk = 2 * i + p

            @pl.when(blk < nblk - 1)
            def _():
                fetch(blk + 1, 1 - p)

            crunch(p)
        return carry

    lax.fori_loop(0, nblk // 2, blk_body, 0)
    plsc.subcore_barrier()
    start = s * 1563 - jnp.maximum(s - 8, 0)

    @pl.when(s < 8)
    def _():
        pltpu.sync_copy(acc.at[pl.ds(start, 1563)],
                        out.at[pl.ds(c * HALF + start, 1563)])

    @pl.when(s >= 8)
    def _():
        pltpu.sync_copy(acc.at[pl.ds(start, 1562)],
                        out.at[pl.ds(c * HALF + start, 1562)])


def _make_den_kernel(E_pad, H):
    nblk = E_pad // (NS * B)
    mesh = plsc.VectorSubcoreMesh(core_axis_name="c", subcore_axis_name="s",
                                  num_cores=NC, num_subcores=NS)
    return pl.kernel(
        functools.partial(_den_body, H, nblk),
        out_type=jax.ShapeDtypeStruct((N, 16), jnp.float32),
        mesh=mesh,
        compiler_params=_SC_PARAMS,
        scratch_types=[
            pltpu.VMEM((3, B), jnp.int32),
            pltpu.VMEM((3, B), jnp.int32),
            pltpu.VMEM((B,), jnp.int32),
            pltpu.VMEM((B,), jnp.int32),
            pltpu.VMEM((B, 16), jnp.float32),
            pltpu.VMEM((B, 16), jnp.float32),
            pltpu.VMEM((B, 16), jnp.float32),
            pltpu.VMEM((B, 16), jnp.float32),
            pltpu.VMEM((B, 16), jnp.float32),
            pltpu.SemaphoreType.DMA,
            pltpu.SemaphoreType.DMA,
            pltpu.VMEM_SHARED((DACC_ROWS, 16), jnp.float32),
        ],
    )


def _fold4(W, a_s, a_d):
    F = W.shape[1]
    eyeH = jnp.eye(4, dtype=jnp.float32)
    Ss = (a_s[:, :, None] * eyeH[:, None, :]).reshape(F, 4)
    Sd = (a_d[:, :, None] * eyeH[:, None, :]).reshape(F, 4)
    Wes = jnp.pad(W @ Ss, ((0, 0), (0, 12)))
    Wed = jnp.pad(W @ Sd, ((0, 0), (0, 12)))
    Wstk = jnp.stack([jnp.concatenate([Wes, W[:, 0:32]], axis=1),
                      jnp.concatenate([Wes, W[:, 32:64]], axis=1)])
    bias = jnp.zeros((2, 1, 48), jnp.float32)
    return Wstk, bias, Wed, Wes


def _fold1(W, a_s, a_d, Fc):
    fin = W.shape[0]
    Wes = (W @ a_s.T) @ jnp.ones((1, 16), jnp.float32)
    Wed = (W @ a_d.T) @ jnp.ones((1, 16), jnp.float32)
    hpad = jnp.pad(W, ((0, 0), (0, Fc - W.shape[1])))
    Wstk = jnp.stack([jnp.concatenate([Wes, hpad], axis=1),
                      jnp.concatenate([Wes, jnp.zeros((fin, Fc), jnp.float32)],
                                      axis=1)])
    maskrow = jnp.concatenate(
        [jnp.zeros((1, 16), jnp.float32),
         (lax.iota(jnp.int32, Fc) < 1)[None].astype(jnp.float32)], axis=1)
    bias = jnp.stack([jnp.zeros((1, 16 + Fc), jnp.float32), maskrow])
    return Wstk, bias, Wed, Wes


def _dm4():
    Dm = jnp.kron(jnp.eye(4, dtype=jnp.float32), jnp.ones((1, 16), jnp.float32))
    return jnp.pad(Dm, ((0, 12), (0, 0)))


def kernel(x, edge_index, W1, as1, ad1, b1, W2, as2, ad2, b2, W3, as3, ad3, b3,
           W4, as4, ad4, b4, g1, bt1, g2, bt2, g3, bt3, mask1, mask2):
    src, dst = edge_index[0], edge_index[1]
    nblk = (E + NS * B - 1) // (NS * B)
    nblk += (-nblk) % 4
    e_pad = nblk * NS * B
    pad = e_pad - E
    src_p = jnp.concatenate([src, jnp.zeros((pad,), jnp.int32)])
    dst_p = jnp.concatenate([dst, jnp.full((pad,), N, jnp.int32)])
    idx3 = jnp.stack([src_p, src_p + N, dst_p])
    z32 = jnp.zeros((3128, 32), jnp.float32)
    z16n = jnp.zeros((3128, 16), jnp.float32)
    z16d = jnp.zeros((1563, 16), jnp.float32)

    num64 = _make_num_kernel(e_pad, 32, 16)
    den4 = _make_den_kernel(e_pad, 4)
    num32 = _make_num_kernel(e_pad, 32, 32)
    num16 = _make_num_kernel(e_pad, 16, 16)

    Wstk, bias, Wed, Wes = _fold4(W1, as1, ad1)
    ts, td, te = _tables(x, Wstk, bias, Wed, Wes)
    numarr = num64(ts, td, idx3, z32)
    den = den4(te, td, idx3, z16d)
    o, st = _post_stats(numarr, den, 0, _dm4(), b1.reshape(1, -1), 64, 32)

    Wstk, bias, Wed, Wes = _fold4(W2, as2, ad2)
    ts, td, te = _norm_tables(o, st, g1.reshape(1, -1), bt1.reshape(1, -1),
                              mask1, Wstk, bias, Wed, Wes, True)
    numarr = num64(ts, td, idx3, z32)
    den = den4(te, td, idx3, z16d)
    o, st = _post_stats(numarr, den, 0, _dm4(), b2.reshape(1, -1), 64, 32)

    Wstk, bias, Wed, Wes = _fold1(W3, as3, ad3, 32)
    ts, td, te = _norm_tables(o, st, g2.reshape(1, -1), bt2.reshape(1, -1),
                              mask2, Wstk, bias, Wed, Wes, True)
    numarr = num32(ts, td, idx3, z32)
    dm3 = jnp.zeros((32, 32), jnp.float32).at[0, :].set(1.0)
    o, st = _post_stats(numarr, numarr, NB, dm3, b3.reshape(1, -1), 32, 32)

    W4p = jnp.pad(W4, ((0, 0), (0, 15)))
    as4p = jnp.pad(as4, ((0, 0), (0, 15)))
    ad4p = jnp.pad(ad4, ((0, 0), (0, 15)))
    Wstk, bias, Wed, Wes = _fold1(W4p, as4p, ad4p, 16)
    ts, td, te = _norm_tables(o, st, g3.reshape(1, -1), bt3.reshape(1, -1), o,
                              Wstk, bias, Wed, Wes, False)
    numarr = num16(ts, td, idx3, z16n)

    numrow = numarr[0:N, 0:1].reshape(1, N)
    denrow = numarr[N:2 * N, 0:1].reshape(1, N)
    logits, prob = _final(numrow, denrow, b4.reshape(1, 1))
    return (logits, prob)
